# Initial kernel scaffold; baseline (speedup 1.0000x reference)
import jax
import jax.numpy as jnp
from jax.experimental import pallas as pl


def kernel(xs, xt, edge_index, Wl1, bl1, Wr1, Wl2, bl2, Wr2, g1, b1, g2, b2, Wep, bep, Wout, bout):
    raise NotImplementedError("write your pallas kernel here")


# speedup vs baseline: 10.2323x; 10.2323x over previous
import functools

import jax
import jax.numpy as jnp
import numpy as np
from jax import lax
from jax.experimental import pallas as pl
from jax.experimental.pallas import tpu as pltpu
from jax.experimental.pallas import tpu_sc as plsc

_N = 100000
_E = 1600000
_NS = 16
_NP = 100352
_EP = 1605632
_ROWS_PER_TILE = _NP // _NS
_CHUNKS = _EP // (_NS * 1024)
_INV_SQRT_BN = float(1.0 / np.sqrt(1.0 + 1e-5))
_BK = 2000


def _agg_body(T, src2, dst3, zrows, out, sidx, didx, rows, acc, gsem, ssem):
    c = lax.axis_index("c")
    s = lax.axis_index("s")
    base = s * _ROWS_PER_TILE

    pltpu.sync_copy(zrows, acc.at[pl.ds(base, _ROWS_PER_TILE)])
    plsc.subcore_barrier()

    ebase = s * (_EP // _NS // 128)

    def body(t, carry):
        r0 = ebase + t * 8
        pltpu.sync_copy(src2.at[c, pl.ds(r0, 8)], sidx)
        pltpu.sync_copy(dst3.at[pl.ds(r0, 8)], didx)
        gathers = []
        for j in range(8):
            gathers.append(pltpu.async_copy(T.at[sidx.at[j]], rows.at[j], gsem))
        for j in range(8):
            gathers[j].wait()
        scatters = []
        for j in range(8):
            scatters.append(
                pltpu.async_copy(rows.at[j], acc.at[didx.at[j]], ssem, add=True))
        for j in range(8):
            scatters[j].wait()
        return carry

    lax.fori_loop(0, _CHUNKS, body, 0)
    plsc.subcore_barrier()

    pltpu.sync_copy(acc.at[pl.ds(base, _ROWS_PER_TILE)],
                    out.at[c, pl.ds(base, _ROWS_PER_TILE)])


@functools.cache
def _make_agg():
    return pl.kernel(
        _agg_body,
        out_type=jax.ShapeDtypeStruct((2, _NP, 16), jnp.float32),
        mesh=plsc.VectorSubcoreMesh(core_axis_name="c", subcore_axis_name="s"),
        compiler_params=pltpu.CompilerParams(use_tc_tiling_on_sc=False),
        scratch_types=[
            pltpu.VMEM((8, 128), jnp.int32),
            pltpu.VMEM((8, 128), jnp.int32),
            pltpu.VMEM((8, 128, 16), jnp.float32),
            pltpu.VMEM_SHARED((_NP, 16), jnp.float32),
            pltpu.SemaphoreType.DMA,
            pltpu.SemaphoreType.DMA,
        ],
    )


def _agg(T, src2, dst3, zrows):
    return _make_agg()(T, src2, dst3, zrows)


def _dense1_body(x_ref, sa_ref, sb_ref, wl1aT, wl1bT, wr1T, bl1, g1, b1,
                 wl2T, wr2T, bl2, y2a_ref, y2b_ref, r2_ref):
    sa = sa_ref[...]
    sb = sb_ref[...]
    dinv = 1.0 / jnp.maximum(sb[:, 4:5], 1.0)
    zm = jnp.dot(sa * dinv, wl1aT[...], preferred_element_type=jnp.float32)
    zm = zm + jnp.dot(sb[:, 0:4] * dinv, wl1bT[...],
                      preferred_element_type=jnp.float32)
    z = zm + jnp.dot(x_ref[...], wr1T[...],
                     preferred_element_type=jnp.float32) + bl1[...]
    h = jnp.maximum(z * (g1[...] * _INV_SQRT_BN) + b1[...], 0.0)
    y2 = jnp.dot(h, wl2T[...], preferred_element_type=jnp.float32)
    y2a_ref[...] = y2[:, :16]
    y2b_ref[...] = y2[:, 16:]
    r2_ref[...] = jnp.dot(h, wr2T[...],
                          preferred_element_type=jnp.float32) + bl2[...]


def _dense2_body(sa_ref, sb_ref, s1b_ref, r2_ref, g2, b2, wep, bep, wout,
                 bout, out_ref):
    dinv = 1.0 / jnp.maximum(s1b_ref[:, 4:5], 1.0)
    z2 = jnp.concatenate([sa_ref[...], sb_ref[...]], axis=1) * dinv + r2_ref[...]
    h2 = jnp.maximum(z2 * (g2[...] * _INV_SQRT_BN) + b2[...], 0.0)
    logit = jnp.mean(h2, axis=1, keepdims=True)
    embed = jnp.maximum(logit * wep[...] + bep[...], 0.0)
    out_ref[...] = jnp.sum(embed * wout[...], axis=1, keepdims=True) + bout[...]


def _row_spec(d):
    return pl.BlockSpec((_BK, d), lambda i: (i, 0))


def _full_spec(r, c):
    return pl.BlockSpec((r, c), lambda i: (0, 0))


def _dense1(x, sa, sb, wl1aT, wl1bT, wr1T, bl1, g1, b1, wl2T, wr2T, bl2):
    grid = (_N // _BK,)
    return pl.pallas_call(
        _dense1_body,
        grid=grid,
        in_specs=[
            _row_spec(20), _row_spec(16), _row_spec(16),
            _full_spec(16, 64), _full_spec(4, 64), _full_spec(20, 64),
            _full_spec(1, 64), _full_spec(1, 64), _full_spec(1, 64),
            _full_spec(64, 32), _full_spec(64, 32), _full_spec(1, 32),
        ],
        out_specs=[_row_spec(16), _row_spec(16), _row_spec(32)],
        out_shape=[
            jax.ShapeDtypeStruct((_N, 16), jnp.float32),
            jax.ShapeDtypeStruct((_N, 16), jnp.float32),
            jax.ShapeDtypeStruct((_N, 32), jnp.float32),
        ],
    )(x, sa, sb, wl1aT, wl1bT, wr1T, bl1, g1, b1, wl2T, wr2T, bl2)


def _dense2(sa2, sb2, s1b, r2, g2, b2, wepT, bep, wout, bout):
    grid = (_N // _BK,)
    return pl.pallas_call(
        _dense2_body,
        grid=grid,
        in_specs=[
            _row_spec(16), _row_spec(16), _row_spec(16), _row_spec(32),
            _full_spec(1, 32), _full_spec(1, 32), _full_spec(1, 32),
            _full_spec(1, 32), _full_spec(1, 32), _full_spec(1, 1),
        ],
        out_specs=[_row_spec(1)],
        out_shape=[jax.ShapeDtypeStruct((_N, 1), jnp.float32)],
    )(sa2, sb2, s1b, r2, g2, b2, wepT, bep, wout, bout)


def kernel(xs, xt, edge_index, Wl1, bl1, Wr1, Wl2, bl2, Wr2, g1, b1, g2, b2,
           Wep, bep, Wout, bout):
    x = jnp.concatenate([xs, xt], axis=1)
    npad = _NP - _N
    zpadn = jnp.zeros((npad, 16), jnp.float32)
    t1a = x[:, :16]
    t1b = jnp.concatenate(
        [x[:, 16:20], jnp.ones((_N, 1), jnp.float32),
         jnp.zeros((_N, 11), jnp.float32)], axis=1)
    T1 = jnp.concatenate([t1a, zpadn, t1b, zpadn], axis=0)

    src = edge_index[0]
    dst = edge_index[1]
    epad = _EP - _E
    srcp = jnp.concatenate([src, jnp.full((epad,), _N, jnp.int32)])
    dstp = jnp.concatenate([dst, jnp.full((epad,), _N, jnp.int32)])
    src2 = jnp.stack([srcp, srcp + _NP]).reshape(2, _EP // 128, 128)
    dst3 = dstp.reshape(_EP // 128, 128)
    zrows = jnp.zeros((_ROWS_PER_TILE, 16), jnp.float32)

    s1 = _agg(T1, src2, dst3, zrows)
    s1a = s1[0, :_N]
    s1b = s1[1, :_N]

    y2a, y2b, r2 = _dense1(
        x, s1a, s1b,
        Wl1[:, :16].T, Wl1[:, 16:20].T, Wr1.T,
        bl1.reshape(1, 64), g1.reshape(1, 64), b1.reshape(1, 64),
        Wl2.T, Wr2.T, bl2.reshape(1, 32))

    T2 = jnp.concatenate([y2a, zpadn, y2b, zpadn], axis=0)
    s2 = _agg(T2, src2, dst3, zrows)

    out2d = _dense2(
        s2[0, :_N], s2[1, :_N], s1b, r2,
        g2.reshape(1, 32), b2.reshape(1, 32),
        Wep.T, bep.reshape(1, 32), Wout, bout.reshape(1, 1))[0]
    return out2d[:, 0]


# speedup vs baseline: 11.4889x; 1.1228x over previous
import functools

import jax
import jax.numpy as jnp
import numpy as np
from jax import lax
from jax.experimental import pallas as pl
from jax.experimental.pallas import tpu as pltpu
from jax.experimental.pallas import tpu_sc as plsc

_N = 100000
_E = 1600000
_NS = 16
_NP = 100096
_K = 6
_CHUNKS = 131
_EP = _NS * _K * 128 * _CHUNKS
_ROWS_PER_TILE = _NP // _NS
_INV_SQRT_BN = float(1.0 / np.sqrt(1.0 + 1e-5))
_BK = 2000


def _agg_body(T, src2, dst3, zrows, out, sidx, didx, rows, acc, gsem, ssem):
    c = lax.axis_index("c")
    s = lax.axis_index("s")
    base = s * _ROWS_PER_TILE

    pltpu.sync_copy(zrows, acc.at[pl.ds(base, _ROWS_PER_TILE)])
    plsc.subcore_barrier()

    ebase = s * (_EP // _NS // 128)

    def load_idx(slot, t):
        r0 = ebase + t * _K
        pltpu.sync_copy(src2.at[c, pl.ds(r0, _K)], sidx.at[slot])
        pltpu.sync_copy(dst3.at[pl.ds(r0, _K)], didx.at[slot])

    def fire_gathers(slot):
        for j in range(_K):
            pltpu.async_copy(T.at[sidx.at[slot, j]], rows.at[slot, j], gsem)

    def wait_gathers(slot):
        for j in range(_K):
            pltpu.make_async_copy(T.at[sidx.at[slot, j]], rows.at[slot, j],
                                  gsem).wait()

    def fire_scatters(slot):
        for j in range(_K):
            pltpu.async_copy(rows.at[slot, j], acc.at[didx.at[slot, j]], ssem,
                             add=True)

    def wait_scatters(slot):
        for j in range(_K):
            pltpu.make_async_copy(rows.at[slot, j], acc.at[didx.at[slot, j]],
                                  ssem).wait()

    load_idx(0, 0)
    fire_gathers(0)

    def body(p, carry):
        @pl.when(p >= 1)
        def _():
            wait_scatters(1)

        load_idx(1, 2 * p + 1)
        fire_gathers(1)
        wait_gathers(0)
        fire_scatters(0)
        wait_scatters(0)
        load_idx(0, 2 * p + 2)
        fire_gathers(0)
        wait_gathers(1)
        fire_scatters(1)
        return carry

    lax.fori_loop(0, (_CHUNKS - 1) // 2, body, 0, unroll=False)

    wait_scatters(1)
    wait_gathers(0)
    fire_scatters(0)
    wait_scatters(0)

    plsc.subcore_barrier()

    pltpu.sync_copy(acc.at[pl.ds(base, _ROWS_PER_TILE)],
                    out.at[c, pl.ds(base, _ROWS_PER_TILE)])


@functools.cache
def _make_agg():
    return pl.kernel(
        _agg_body,
        out_type=jax.ShapeDtypeStruct((2, _NP, 16), jnp.float32),
        mesh=plsc.VectorSubcoreMesh(core_axis_name="c", subcore_axis_name="s"),
        compiler_params=pltpu.CompilerParams(use_tc_tiling_on_sc=False),
        scratch_types=[
            pltpu.VMEM((2, _K, 128), jnp.int32),
            pltpu.VMEM((2, _K, 128), jnp.int32),
            pltpu.VMEM((2, _K, 128, 16), jnp.float32),
            pltpu.VMEM_SHARED((_NP, 16), jnp.float32),
            pltpu.SemaphoreType.DMA,
            pltpu.SemaphoreType.DMA,
        ],
    )


def _agg(T, src2, dst3, zrows):
    return _make_agg()(T, src2, dst3, zrows)


def _dense1_body(x_ref, sa_ref, sb_ref, wl1aT, wl1bT, wr1T, bl1, g1, b1,
                 wl2T, wr2T, bl2, y2a_ref, y2b_ref, r2_ref):
    sa = sa_ref[...]
    sb = sb_ref[...]
    dinv = 1.0 / jnp.maximum(sb[:, 4:5], 1.0)
    zm = jnp.dot(sa * dinv, wl1aT[...], preferred_element_type=jnp.float32)
    zm = zm + jnp.dot(sb[:, 0:4] * dinv, wl1bT[...],
                      preferred_element_type=jnp.float32)
    z = zm + jnp.dot(x_ref[...], wr1T[...],
                     preferred_element_type=jnp.float32) + bl1[...]
    h = jnp.maximum(z * (g1[...] * _INV_SQRT_BN) + b1[...], 0.0)
    y2 = jnp.dot(h, wl2T[...], preferred_element_type=jnp.float32)
    y2a_ref[...] = y2[:, :16]
    y2b_ref[...] = y2[:, 16:]
    r2_ref[...] = jnp.dot(h, wr2T[...],
                          preferred_element_type=jnp.float32) + bl2[...]


def _dense2_body(sa_ref, sb_ref, s1b_ref, r2_ref, g2, b2, wep, bep, wout,
                 bout, out_ref):
    dinv = 1.0 / jnp.maximum(s1b_ref[:, 4:5], 1.0)
    z2 = jnp.concatenate([sa_ref[...], sb_ref[...]], axis=1) * dinv + r2_ref[...]
    h2 = jnp.maximum(z2 * (g2[...] * _INV_SQRT_BN) + b2[...], 0.0)
    logit = jnp.mean(h2, axis=1, keepdims=True)
    embed = jnp.maximum(logit * wep[...] + bep[...], 0.0)
    out_ref[...] = jnp.sum(embed * wout[...], axis=1, keepdims=True) + bout[...]


def _row_spec(d):
    return pl.BlockSpec((_BK, d), lambda i: (i, 0))


def _full_spec(r, c):
    return pl.BlockSpec((r, c), lambda i: (0, 0))


def _dense1(x, sa, sb, wl1aT, wl1bT, wr1T, bl1, g1, b1, wl2T, wr2T, bl2):
    grid = (_N // _BK,)
    return pl.pallas_call(
        _dense1_body,
        grid=grid,
        in_specs=[
            _row_spec(20), _row_spec(16), _row_spec(16),
            _full_spec(16, 64), _full_spec(4, 64), _full_spec(20, 64),
            _full_spec(1, 64), _full_spec(1, 64), _full_spec(1, 64),
            _full_spec(64, 32), _full_spec(64, 32), _full_spec(1, 32),
        ],
        out_specs=[_row_spec(16), _row_spec(16), _row_spec(32)],
        out_shape=[
            jax.ShapeDtypeStruct((_N, 16), jnp.float32),
            jax.ShapeDtypeStruct((_N, 16), jnp.float32),
            jax.ShapeDtypeStruct((_N, 32), jnp.float32),
        ],
    )(x, sa, sb, wl1aT, wl1bT, wr1T, bl1, g1, b1, wl2T, wr2T, bl2)


def _dense2(sa2, sb2, s1b, r2, g2, b2, wepT, bep, wout, bout):
    grid = (_N // _BK,)
    return pl.pallas_call(
        _dense2_body,
        grid=grid,
        in_specs=[
            _row_spec(16), _row_spec(16), _row_spec(16), _row_spec(32),
            _full_spec(1, 32), _full_spec(1, 32), _full_spec(1, 32),
            _full_spec(1, 32), _full_spec(1, 32), _full_spec(1, 1),
        ],
        out_specs=[_row_spec(1)],
        out_shape=[jax.ShapeDtypeStruct((_N, 1), jnp.float32)],
    )(sa2, sb2, s1b, r2, g2, b2, wepT, bep, wout, bout)


def kernel(xs, xt, edge_index, Wl1, bl1, Wr1, Wl2, bl2, Wr2, g1, b1, g2, b2,
           Wep, bep, Wout, bout):
    x = jnp.concatenate([xs, xt], axis=1)
    npad = _NP - _N
    zpadn = jnp.zeros((npad, 16), jnp.float32)
    t1a = x[:, :16]
    t1b = jnp.concatenate(
        [x[:, 16:20], jnp.ones((_N, 1), jnp.float32),
         jnp.zeros((_N, 11), jnp.float32)], axis=1)
    T1 = jnp.concatenate([t1a, zpadn, t1b, zpadn], axis=0)

    src = edge_index[0]
    dst = edge_index[1]
    epad = _EP - _E
    srcp = jnp.concatenate([src, jnp.full((epad,), _N, jnp.int32)])
    dstp = jnp.concatenate([dst, jnp.full((epad,), _N, jnp.int32)])
    src2 = jnp.stack([srcp, srcp + _NP]).reshape(2, _EP // 128, 128)
    dst3 = dstp.reshape(_EP // 128, 128)
    zrows = jnp.zeros((_ROWS_PER_TILE, 16), jnp.float32)

    s1 = _agg(T1, src2, dst3, zrows)
    s1a = s1[0, :_N]
    s1b = s1[1, :_N]

    y2a, y2b, r2 = _dense1(
        x, s1a, s1b,
        Wl1[:, :16].T, Wl1[:, 16:20].T, Wr1.T,
        bl1.reshape(1, 64), g1.reshape(1, 64), b1.reshape(1, 64),
        Wl2.T, Wr2.T, bl2.reshape(1, 32))

    T2 = jnp.concatenate([y2a, zpadn, y2b, zpadn], axis=0)
    s2 = _agg(T2, src2, dst3, zrows)

    out2d = _dense2(
        s2[0, :_N], s2[1, :_N], s1b, r2,
        g2.reshape(1, 32), b2.reshape(1, 32),
        Wep.T, bep.reshape(1, 32), Wout, bout.reshape(1, 1))[0]
    return out2d[:, 0]


# speedup vs baseline: 13.2129x; 1.1501x over previous
import functools

import jax
import jax.numpy as jnp
import numpy as np
from jax import lax
from jax.experimental import pallas as pl
from jax.experimental.pallas import tpu as pltpu
from jax.experimental.pallas import tpu_sc as plsc

_N = 100000
_E = 1600000
_NS = 16
_NP = 100096
_K = 6
_CHUNKS = 131
_EP = _NS * _K * 128 * _CHUNKS
_ROWS_PER_TILE = _NP // _NS
_INV_SQRT_BN = float(1.0 / np.sqrt(1.0 + 1e-5))
_BK = 2000


def _agg_body(T, srcR, dstR, zrows, out, sidx, didx, rows, acc, gsem, ssem):
    c = lax.axis_index("c")
    s = lax.axis_index("s")
    base = s * _ROWS_PER_TILE
    tab = T.at[c]

    pltpu.sync_copy(zrows, acc.at[pl.ds(base, _ROWS_PER_TILE)])
    plsc.subcore_barrier()

    ebase = s * (_EP // _NS // 128)

    def load_idx(slot, t):
        r0 = ebase + t * _K
        pltpu.sync_copy(srcR.at[pl.ds(r0, _K)], sidx.at[slot])
        pltpu.sync_copy(dstR.at[pl.ds(r0, _K)], didx.at[slot])

    def fire_gathers(slot):
        for j in range(_K):
            pltpu.async_copy(tab.at[sidx.at[slot, j]], rows.at[slot, j], gsem)

    def wait_gathers(slot):
        for j in range(_K):
            pltpu.make_async_copy(tab.at[sidx.at[slot, j]], rows.at[slot, j],
                                  gsem).wait()

    def fire_scatters(slot):
        for j in range(_K):
            pltpu.async_copy(rows.at[slot, j], acc.at[didx.at[slot, j]], ssem,
                             add=True)

    def wait_scatters(slot):
        for j in range(_K):
            pltpu.make_async_copy(rows.at[slot, j], acc.at[didx.at[slot, j]],
                                  ssem).wait()

    load_idx(0, 0)
    fire_gathers(0)

    def body(p, carry):
        @pl.when(p >= 1)
        def _():
            wait_scatters(1)

        load_idx(1, 2 * p + 1)
        fire_gathers(1)
        wait_gathers(0)
        fire_scatters(0)
        wait_scatters(0)
        load_idx(0, 2 * p + 2)
        fire_gathers(0)
        wait_gathers(1)
        fire_scatters(1)
        return carry

    lax.fori_loop(0, (_CHUNKS - 1) // 2, body, 0, unroll=False)

    wait_scatters(1)
    wait_gathers(0)
    fire_scatters(0)
    wait_scatters(0)

    plsc.subcore_barrier()

    pltpu.sync_copy(acc.at[pl.ds(base, _ROWS_PER_TILE)],
                    out.at[c, pl.ds(base, _ROWS_PER_TILE)])


@functools.cache
def _make_agg():
    return pl.kernel(
        _agg_body,
        out_type=jax.ShapeDtypeStruct((2, _NP, 16), jnp.float32),
        mesh=plsc.VectorSubcoreMesh(core_axis_name="c", subcore_axis_name="s"),
        compiler_params=pltpu.CompilerParams(use_tc_tiling_on_sc=False),
        scratch_types=[
            pltpu.VMEM((2, _K, 128), jnp.int32),
            pltpu.VMEM((2, _K, 128), jnp.int32),
            pltpu.VMEM((2, _K, 128, 16), jnp.float32),
            pltpu.VMEM_SHARED((_NP, 16), jnp.float32),
            pltpu.SemaphoreType.DMA,
            pltpu.SemaphoreType.DMA,
        ],
    )


def _agg(T, srcR, dstR, zrows):
    return _make_agg()(T, srcR, dstR, zrows)


def _prep_body(xs_ref, xt_ref, t1_ref):
    xs = xs_ref[...]
    xt = xt_ref[...]
    t1_ref[0] = jnp.concatenate([xs, xt[:, :10]], axis=1)
    t1_ref[1] = jnp.concatenate(
        [xt[:, 10:14], jnp.ones((_BK, 1), jnp.float32),
         jnp.zeros((_BK, 11), jnp.float32)], axis=1)


def _dense1_body(xs_ref, xt_ref, s1_ref, wl1aT, wl1bT, wr1T, bl1, g1, b1,
                 wl2T, wr2T, bl2, t2_ref, r2_ref):
    sa = s1_ref[0]
    sb = s1_ref[1]
    dinv = 1.0 / jnp.maximum(sb[:, 4:5], 1.0)
    zm = jnp.dot(sa * dinv, wl1aT[...], preferred_element_type=jnp.float32)
    zm = zm + jnp.dot(sb[:, 0:4] * dinv, wl1bT[...],
                      preferred_element_type=jnp.float32)
    x = jnp.concatenate([xs_ref[...], xt_ref[...]], axis=1)
    z = zm + jnp.dot(x, wr1T[...],
                     preferred_element_type=jnp.float32) + bl1[...]
    h = jnp.maximum(z * (g1[...] * _INV_SQRT_BN) + b1[...], 0.0)
    y2 = jnp.dot(h, wl2T[...], preferred_element_type=jnp.float32)
    t2_ref[0] = y2[:, :16]
    t2_ref[1] = y2[:, 16:]
    r2_ref[...] = jnp.dot(h, wr2T[...],
                          preferred_element_type=jnp.float32) + bl2[...]


def _dense2_body(s2_ref, s1b_ref, r2_ref, g2, b2, wep, bep, wout, bout,
                 out_ref):
    dinv = 1.0 / jnp.maximum(s1b_ref[0][:, 4:5], 1.0)
    z2 = jnp.concatenate([s2_ref[0], s2_ref[1]], axis=1) * dinv + r2_ref[...]
    h2 = jnp.maximum(z2 * (g2[...] * _INV_SQRT_BN) + b2[...], 0.0)
    logit = jnp.mean(h2, axis=1, keepdims=True)
    embed = jnp.maximum(logit * wep[...] + bep[...], 0.0)
    out_ref[...] = jnp.sum(embed * wout[...], axis=1, keepdims=True) + bout[...]


def _row_spec(d):
    return pl.BlockSpec((_BK, d), lambda i: (i, 0))


def _tab_spec():
    return pl.BlockSpec((2, _BK, 16), lambda i: (0, i, 0))


def _full_spec(r, c):
    return pl.BlockSpec((r, c), lambda i: (0, 0))


_GRID = (_N // _BK,)


def _prep(xs, xt):
    return pl.pallas_call(
        _prep_body,
        grid=_GRID,
        in_specs=[_row_spec(6), _row_spec(14)],
        out_specs=[_tab_spec()],
        out_shape=[jax.ShapeDtypeStruct((2, _NP, 16), jnp.float32)],
    )(xs, xt)[0]


def _dense1(xs, xt, s1, wl1aT, wl1bT, wr1T, bl1, g1, b1, wl2T, wr2T, bl2):
    return pl.pallas_call(
        _dense1_body,
        grid=_GRID,
        in_specs=[
            _row_spec(6), _row_spec(14), _tab_spec(),
            _full_spec(16, 64), _full_spec(4, 64), _full_spec(20, 64),
            _full_spec(1, 64), _full_spec(1, 64), _full_spec(1, 64),
            _full_spec(64, 32), _full_spec(64, 32), _full_spec(1, 32),
        ],
        out_specs=[_tab_spec(), _row_spec(32)],
        out_shape=[
            jax.ShapeDtypeStruct((2, _NP, 16), jnp.float32),
            jax.ShapeDtypeStruct((_N, 32), jnp.float32),
        ],
    )(xs, xt, s1, wl1aT, wl1bT, wr1T, bl1, g1, b1, wl2T, wr2T, bl2)


def _dense2(s2, s1, r2, g2, b2, wepT, bep, wout, bout):
    return pl.pallas_call(
        _dense2_body,
        grid=_GRID,
        in_specs=[
            _tab_spec(),
            pl.BlockSpec((1, _BK, 16), lambda i: (1, i, 0)),
            _row_spec(32),
            _full_spec(1, 32), _full_spec(1, 32), _full_spec(1, 32),
            _full_spec(1, 32), _full_spec(1, 32), _full_spec(1, 1),
        ],
        out_specs=[_row_spec(1)],
        out_shape=[jax.ShapeDtypeStruct((_N, 1), jnp.float32)],
    )(s2, s1, r2, g2, b2, wepT, bep, wout, bout)[0]


def kernel(xs, xt, edge_index, Wl1, bl1, Wr1, Wl2, bl2, Wr2, g1, b1, g2, b2,
           Wep, bep, Wout, bout):
    src = edge_index[0]
    dst = edge_index[1]
    epad = _EP - _E
    srcR = jnp.concatenate(
        [src, jnp.full((epad,), _N, jnp.int32)]).reshape(_EP // 128, 128)
    dstR = jnp.concatenate(
        [dst, jnp.full((epad,), _N, jnp.int32)]).reshape(_EP // 128, 128)
    zrows = jnp.zeros((_ROWS_PER_TILE, 16), jnp.float32)

    T1 = _prep(xs, xt)
    s1 = _agg(T1, srcR, dstR, zrows)

    T2, r2 = _dense1(
        xs, xt, s1,
        Wl1[:, :16].T, Wl1[:, 16:20].T, Wr1.T,
        bl1.reshape(1, 64), g1.reshape(1, 64), b1.reshape(1, 64),
        Wl2.T, Wr2.T, bl2.reshape(1, 32))

    s2 = _agg(T2, srcR, dstR, zrows)

    out2d = _dense2(
        s2, s1, r2,
        g2.reshape(1, 32), b2.reshape(1, 32),
        Wep.T, bep.reshape(1, 32), Wout, bout.reshape(1, 1))
    return out2d[:, 0]


# speedup vs baseline: 16.2694x; 1.2313x over previous
import functools

import jax
import jax.numpy as jnp
import numpy as np
from jax import lax
from jax.experimental import pallas as pl
from jax.experimental.pallas import tpu as pltpu
from jax.experimental.pallas import tpu_sc as plsc

_N = 100000
_E = 1600000
_NS = 16
_NP = 100096
_NPR = _NP * 16 // 128
_K = 6
_CHUNKS = 131
_EP = _NS * _K * 128 * _CHUNKS
_ROWS_PER_TILE = _NP // _NS
_INV_SQRT_BN = float(1.0 / np.sqrt(1.0 + 1e-5))
_BK = 4096
_BKR = _BK * 16 // 128


def _agg_body(T, srcR, dstR, zrows, out, sidx, didx, rows, acc, gsem, ssem):
    c = lax.axis_index("c")
    s = lax.axis_index("s")
    base = s * _ROWS_PER_TILE
    tab = T.at[c]

    pltpu.sync_copy(zrows, acc.at[pl.ds(base, _ROWS_PER_TILE)])
    plsc.subcore_barrier()

    ebase = s * (_EP // _NS // 128)

    def load_idx(slot, t):
        r0 = ebase + t * _K
        pltpu.sync_copy(srcR.at[pl.ds(r0, _K)], sidx.at[slot])
        pltpu.sync_copy(dstR.at[pl.ds(r0, _K)], didx.at[slot])

    def fire_gathers(slot):
        for j in range(_K):
            pltpu.async_copy(tab.at[sidx.at[slot, j]], rows.at[slot, j], gsem)

    def wait_gathers(slot):
        for j in range(_K):
            pltpu.make_async_copy(tab.at[sidx.at[slot, j]], rows.at[slot, j],
                                  gsem).wait()

    def fire_scatters(slot):
        for j in range(_K):
            pltpu.async_copy(rows.at[slot, j], acc.at[didx.at[slot, j]], ssem,
                             add=True)

    def wait_scatters(slot):
        for j in range(_K):
            pltpu.make_async_copy(rows.at[slot, j], acc.at[didx.at[slot, j]],
                                  ssem).wait()

    load_idx(0, 0)
    fire_gathers(0)

    def body(p, carry):
        @pl.when(p >= 1)
        def _():
            wait_scatters(1)

        load_idx(1, 2 * p + 1)
        fire_gathers(1)
        wait_gathers(0)
        fire_scatters(0)
        wait_scatters(0)
        load_idx(0, 2 * p + 2)
        fire_gathers(0)
        wait_gathers(1)
        fire_scatters(1)
        return carry

    lax.fori_loop(0, (_CHUNKS - 1) // 2, body, 0, unroll=False)

    wait_scatters(1)
    wait_gathers(0)
    fire_scatters(0)
    wait_scatters(0)

    plsc.subcore_barrier()

    pltpu.sync_copy(acc.at[pl.ds(base, _ROWS_PER_TILE)],
                    out.at[c, pl.ds(base, _ROWS_PER_TILE)])


@functools.cache
def _make_agg():
    return pl.kernel(
        _agg_body,
        out_type=jax.ShapeDtypeStruct((2, _NP, 16), jnp.float32),
        mesh=plsc.VectorSubcoreMesh(core_axis_name="c", subcore_axis_name="s"),
        compiler_params=pltpu.CompilerParams(use_tc_tiling_on_sc=False,
                                             disable_bounds_checks=True),
        scratch_types=[
            pltpu.VMEM((2, _K, 128), jnp.int32),
            pltpu.VMEM((2, _K, 128), jnp.int32),
            pltpu.VMEM((2, _K, 128, 16), jnp.float32),
            pltpu.VMEM_SHARED((_NP, 16), jnp.float32),
            pltpu.SemaphoreType.DMA,
            pltpu.SemaphoreType.DMA,
        ],
    )


def _agg(Tp, srcR, dstR, zrows):
    T = Tp.reshape(2, _NP, 16)
    s = _make_agg()(T, srcR, dstR, zrows)
    return s.reshape(2, _NPR, 128)


def _unpack(xp, w):
    return jnp.concatenate([xp[:, w * n:w * (n + 1)] for n in range(8)],
                           axis=0)


def _pack(x):
    r = x.shape[0] // 8
    return jnp.concatenate([x[n * r:(n + 1) * r] for n in range(8)], axis=1)


def _dense1_body(t1_ref, s1_ref, wl1aT, wl1bT, wr1aT, wr1bT, bl1, g1, b1,
                 wl2T, wr2T, bl2, t2_ref, r2_ref):
    sa = _unpack(s1_ref[0], 16)
    sb = _unpack(s1_ref[1], 16)
    xa = _unpack(t1_ref[0], 16)
    xb = _unpack(t1_ref[1], 16)
    dinv = 1.0 / jnp.maximum(sb[:, 4:5], 1.0)
    z = jnp.dot(sa * dinv, wl1aT[...], preferred_element_type=jnp.float32)
    z = z + jnp.dot(sb[:, 0:4] * dinv, wl1bT[...],
                    preferred_element_type=jnp.float32)
    z = z + jnp.dot(xa, wr1aT[...], preferred_element_type=jnp.float32)
    z = z + jnp.dot(xb[:, 0:4], wr1bT[...],
                    preferred_element_type=jnp.float32)
    z = z + bl1[...]
    h = jnp.maximum(z * (g1[...] * _INV_SQRT_BN) + b1[...], 0.0)
    y2 = jnp.dot(h, wl2T[...], preferred_element_type=jnp.float32)
    t2_ref[0] = _pack(y2[:, :16])
    t2_ref[1] = _pack(y2[:, 16:])
    r2 = jnp.dot(h, wr2T[...], preferred_element_type=jnp.float32) + bl2[...]
    r2_ref[...] = _pack(r2)


def _dense2_body(s2_ref, s1b_ref, r2_ref, g2, b2, wep, bep, wout, bout,
                 out_ref):
    sa = _unpack(s2_ref[0], 16)
    sb = _unpack(s2_ref[1], 16)
    sbd = _unpack(s1b_ref[0], 16)
    r2 = _unpack(r2_ref[...], 32)
    dinv = 1.0 / jnp.maximum(sbd[:, 4:5], 1.0)
    z2 = jnp.concatenate([sa, sb], axis=1) * dinv + r2
    h2 = jnp.maximum(z2 * (g2[...] * _INV_SQRT_BN) + b2[...], 0.0)
    logit = jnp.mean(h2, axis=1, keepdims=True)
    embed = jnp.maximum(logit * wep[...] + bep[...], 0.0)
    res = jnp.sum(embed * wout[...], axis=1, keepdims=True) + bout[...]
    out_ref[...] = _pack(res)


def _tab_spec():
    return pl.BlockSpec((2, _BKR, 128), lambda i: (0, i, 0))


def _full_spec(r, c):
    return pl.BlockSpec((r, c), lambda i: (0, 0))


_GRID = ((_N + _BK - 1) // _BK,)
_NOUT = _GRID[0] * _BKR


def _dense1(t1p, s1p, wl1aT, wl1bT, wr1aT, wr1bT, bl1, g1, b1, wl2T, wr2T,
            bl2):
    return pl.pallas_call(
        _dense1_body,
        grid=_GRID,
        in_specs=[
            _tab_spec(), _tab_spec(),
            _full_spec(16, 64), _full_spec(4, 64),
            _full_spec(16, 64), _full_spec(4, 64),
            _full_spec(1, 64), _full_spec(1, 64), _full_spec(1, 64),
            _full_spec(64, 32), _full_spec(64, 32), _full_spec(1, 32),
        ],
        out_specs=[_tab_spec(),
                   pl.BlockSpec((_BKR, 256), lambda i: (i, 0))],
        out_shape=[
            jax.ShapeDtypeStruct((2, _NPR, 128), jnp.float32),
            jax.ShapeDtypeStruct((_NOUT, 256), jnp.float32),
        ],
    )(t1p, s1p, wl1aT, wl1bT, wr1aT, wr1bT, bl1, g1, b1, wl2T, wr2T, bl2)


def _dense2(s2p, s1p, r2p, g2, b2, wepT, bep, wout, bout):
    return pl.pallas_call(
        _dense2_body,
        grid=_GRID,
        in_specs=[
            _tab_spec(),
            pl.BlockSpec((1, _BKR, 128), lambda i: (1, i, 0)),
            pl.BlockSpec((_BKR, 256), lambda i: (i, 0)),
            _full_spec(1, 32), _full_spec(1, 32), _full_spec(1, 32),
            _full_spec(1, 32), _full_spec(1, 32), _full_spec(1, 1),
        ],
        out_specs=[pl.BlockSpec((_BKR, 8), lambda i: (i, 0))],
        out_shape=[jax.ShapeDtypeStruct((_NOUT, 8), jnp.float32)],
    )(s2p, s1p, r2p, g2, b2, wepT, bep, wout, bout)[0]


def kernel(xs, xt, edge_index, Wl1, bl1, Wr1, Wl2, bl2, Wr2, g1, b1, g2, b2,
           Wep, bep, Wout, bout):
    src = edge_index[0]
    dst = edge_index[1]
    epad = _EP - _E
    srcR = jnp.concatenate(
        [src, jnp.full((epad,), _N, jnp.int32)]).reshape(_EP // 128, 128)
    dstR = jnp.concatenate(
        [dst, jnp.full((epad,), _N, jnp.int32)]).reshape(_EP // 128, 128)
    zrows = jnp.zeros((_ROWS_PER_TILE, 16), jnp.float32)

    npad = ((0, _NP - _N), (0, 0))
    gA = jnp.pad(jnp.concatenate([xs, xt[:, :10]], axis=1), npad)
    gB = jnp.pad(jnp.concatenate(
        [xt[:, 10:14], jnp.ones((_N, 1), jnp.float32),
         jnp.zeros((_N, 11), jnp.float32)], axis=1), npad)
    T1p = jnp.stack([gA, gB]).reshape(2, _NPR, 128)
    s1p = _agg(T1p, srcR, dstR, zrows)

    T2p, r2p = _dense1(
        T1p, s1p,
        Wl1[:, :16].T, Wl1[:, 16:20].T, Wr1[:, :16].T, Wr1[:, 16:20].T,
        bl1.reshape(1, 64), g1.reshape(1, 64), b1.reshape(1, 64),
        Wl2.T, Wr2.T, bl2.reshape(1, 32))

    s2p = _agg(T2p, srcR, dstR, zrows)

    outp = _dense2(
        s2p, s1p, r2p,
        g2.reshape(1, 32), b2.reshape(1, 32),
        Wep.T, bep.reshape(1, 32), Wout, bout.reshape(1, 1))
    return outp.reshape(-1)[:_N]
